# baseline (speedup 1.0000x reference)
import functools

import jax
import jax.numpy as jnp
from jax import lax
from jax.experimental import pallas as pl
from jax.experimental.pallas import tpu as pltpu
from jax.experimental.pallas import tpu_sc as plsc

VOCAB = 1000000
EMBED_DIM = 64
BATCH = 4096
HIST = 200

NUM_CORES = 2
NUM_SUBCORES = 16

B_TOTAL = BATCH * HIST
NW = NUM_CORES * NUM_SUBCORES
B_PER_W = B_TOTAL // NW
CHUNK = 128
NCHUNK = B_PER_W // CHUNK
NBUF = 2

_mesh = plsc.VectorSubcoreMesh(
    core_axis_name="c", subcore_axis_name="s",
    num_cores=NUM_CORES, num_subcores=NUM_SUBCORES,
)


@functools.partial(
    pl.kernel,
    out_type=jax.ShapeDtypeStruct((B_TOTAL, EMBED_DIM), jnp.float32),
    mesh=_mesh,
    scratch_types=[
        pltpu.VMEM((NCHUNK, CHUNK), jnp.int32),
        pltpu.VMEM((NBUF, CHUNK, EMBED_DIM), jnp.float32),
        pltpu.SemaphoreType.DMA,
        pltpu.SemaphoreType.DMA,
        pltpu.SemaphoreType.DMA,
        pltpu.SemaphoreType.DMA,
    ],
    compiler_params=pltpu.CompilerParams(use_tc_tiling_on_sc=False),
)
def _sc_gather(x_hbm, table_hbm, out_hbm, idx_v, rows_v, g0, g1, w0, w1):
    wid = lax.axis_index("s") * NUM_CORES + lax.axis_index("c")
    base = wid * B_PER_W

    gsem = (g0, g1)
    wsem = (w0, w1)

    pltpu.sync_copy(x_hbm.at[pl.ds(wid * NCHUNK, NCHUNK)], idx_v)

    def gather_start(j, buf):
        pltpu.make_async_copy(
            table_hbm.at[idx_v.at[j]], rows_v.at[buf], gsem[buf]
        ).start()

    def gather_wait(buf):
        pltpu.make_async_copy(
            table_hbm.at[idx_v.at[0]], rows_v.at[buf], gsem[buf]
        ).wait()

    def write_start(j, buf):
        pltpu.make_async_copy(
            rows_v.at[buf], out_hbm.at[pl.ds(base + j * CHUNK, CHUNK)],
            wsem[buf],
        ).start()

    def write_wait(buf):
        pltpu.make_async_copy(
            rows_v.at[buf], out_hbm.at[pl.ds(base, CHUNK)], wsem[buf]
        ).wait()

    gather_start(0, 0)
    gather_start(1, 1)
    gather_wait(0)
    write_start(0, 0)

    @pl.loop(1, NCHUNK - 1, step=NBUF)
    def _(j0):
        for b in range(NBUF):
            j = j0 + b
            buf = (1 + b) % NBUF
            nxt = (b) % NBUF
            write_wait(nxt)
            gather_start(j + 1, nxt)
            gather_wait(buf)
            write_start(j, buf)

    gather_wait((NCHUNK - 1) % NBUF)
    write_start(NCHUNK - 1, (NCHUNK - 1) % NBUF)
    write_wait((NCHUNK - 2) % NBUF)
    write_wait((NCHUNK - 1) % NBUF)


def kernel(x, table):
    x2 = x.reshape(NW * NCHUNK, CHUNK)
    out = _sc_gather(x2, table)
    return out.reshape(BATCH, HIST, EMBED_DIM)
